# baseline (speedup 1.0000x reference)
import jax
import jax.numpy as jnp
from jax import lax
from jax.experimental import pallas as pl
from jax.experimental.pallas import tpu as pltpu
from jax.experimental.pallas import tpu_sc as plsc

VDJ_DIM = 64
PAD_DIM = 128
BATCH = 16384
NUM_TABLES = 5
ROWS = BATCH * NUM_TABLES

_NC = 2
_NS = 16
_NW = _NC * _NS
_RPW = ROWS // _NW
_CHUNK = 128
_NCHUNK = _RPW // _CHUNK


def _gather_body(idx_hbm, tab_hbm, out_hbm, idx_v, buf0, buf1, sg, sw):
    wid = lax.axis_index("s") * _NC + lax.axis_index("c")
    base = wid * _RPW
    pltpu.sync_copy(idx_hbm.at[wid], idx_v)
    bufs = (buf0, buf1)
    writes = [None] * _NCHUNK
    for j in range(_NCHUNK):
        buf = bufs[j % 2]
        if j >= 2:
            writes[j - 2].wait()
        pltpu.async_copy(tab_hbm.at[idx_v.at[j]], buf, sg).wait()
        writes[j] = pltpu.async_copy(
            buf, out_hbm.at[pl.ds(base + j * _CHUNK, _CHUNK)], sw)
    writes[_NCHUNK - 2].wait()
    writes[_NCHUNK - 1].wait()


@jax.jit
def _vdj_gather(x, w0, w1, w2, w3, w4):
    tab = jnp.concatenate([w0, w1, w2, w3, w4], axis=0)
    tab = jnp.pad(tab, ((0, 0), (0, PAD_DIM - VDJ_DIM)))
    off = jnp.arange(NUM_TABLES, dtype=jnp.int32) * (tab.shape[0] // NUM_TABLES)
    idx = (x.astype(jnp.int32) + off[None, :]).T.reshape(_NW, _NCHUNK, _CHUNK)

    kern = pl.kernel(
        _gather_body,
        out_type=jax.ShapeDtypeStruct((ROWS, PAD_DIM), jnp.float32),
        mesh=plsc.VectorSubcoreMesh(core_axis_name="c", subcore_axis_name="s"),
        scratch_types=[
            pltpu.VMEM((_NCHUNK, _CHUNK), jnp.int32),
            pltpu.VMEM((_CHUNK, PAD_DIM), jnp.float32),
            pltpu.VMEM((_CHUNK, PAD_DIM), jnp.float32),
            pltpu.SemaphoreType.DMA,
            pltpu.SemaphoreType.DMA,
        ],
    )
    rows = kern(idx, tab)
    rows = rows.reshape(NUM_TABLES, BATCH, PAD_DIM)
    return tuple(rows[t, :, :VDJ_DIM] for t in range(NUM_TABLES))


def kernel(x, W_v_alpha, W_j_alpha, W_v_beta, W_d_beta, W_j_beta):
    return _vdj_gather(x, W_v_alpha, W_j_alpha, W_v_beta, W_d_beta, W_j_beta)


# speedup vs baseline: 1.0197x; 1.0197x over previous
import jax
import jax.numpy as jnp
from jax import lax
from jax.experimental import pallas as pl
from jax.experimental.pallas import tpu as pltpu
from jax.experimental.pallas import tpu_sc as plsc

VDJ_DIM = 64
VOCAB = 1000
VOCAB_PAD = 1024
BATCH = 16384
NUM_TABLES = 5
OUT_ROWS = NUM_TABLES * VDJ_DIM
TAB_WORDS = VDJ_DIM * VOCAB_PAD
T_WORDS = NUM_TABLES * TAB_WORDS

_NC = 2
_NS = 16
_NW = _NC * _NS
_BPW = BATCH // _NW
_HALF = 32
_FILL = T_WORDS // _NS


def _gather_body(xp_hbm, tab_hbm, out_hbm, spm, idx_v, slab0, slab1, sg, sw):
    cid = lax.axis_index("c")
    sid = lax.axis_index("s")
    wid = sid * _NC + cid
    base = wid * _BPW

    pltpu.sync_copy(tab_hbm.at[pl.ds(sid * _FILL, _FILL)],
                    spm.at[pl.ds(sid * _FILL, _FILL)])
    pltpu.sync_copy(xp_hbm.at[:, pl.ds(wid, 1)], idx_v)
    plsc.subcore_barrier()

    slabs = (slab0, slab1)
    writes = [None, None]
    step = 0
    for t in range(NUM_TABLES):
        for h in range(VDJ_DIM // _HALF):
            slab = slabs[step % 2]
            if writes[step % 2] is not None:
                writes[step % 2].wait()
            gathers = []
            for r in range(_HALF):
                d = h * _HALF + r
                row = spm.at[pl.ds(t * TAB_WORDS + d * VOCAB_PAD, VOCAB_PAD)]
                gathers.append(
                    pltpu.async_copy(row.at[idx_v.at[t, 0]],
                                     slab.at[r, 0], sg))
            for g in gathers:
                g.wait()
            writes[step % 2] = pltpu.async_copy(
                slab,
                out_hbm.at[pl.ds(t * VDJ_DIM + h * _HALF, _HALF),
                           pl.ds(wid, 1)],
                sw)
            step += 1
    writes[0].wait()
    writes[1].wait()


@jax.jit
def _vdj_gather(x, w0, w1, w2, w3, w4):
    xp = x.astype(jnp.int32).T.reshape(NUM_TABLES, _NW, _BPW)
    pad = lambda w: jnp.pad(w.T, ((0, 0), (0, VOCAB_PAD - VOCAB))).reshape(-1)
    tab = jnp.concatenate([pad(w) for w in (w0, w1, w2, w3, w4)])

    kern = pl.kernel(
        _gather_body,
        out_type=jax.ShapeDtypeStruct((OUT_ROWS, _NW, _BPW), jnp.float32),
        mesh=plsc.VectorSubcoreMesh(core_axis_name="c", subcore_axis_name="s"),
        scratch_types=[
            pltpu.VMEM_SHARED((T_WORDS,), jnp.float32),
            pltpu.VMEM((NUM_TABLES, 1, _BPW), jnp.int32),
            pltpu.VMEM((_HALF, 1, _BPW), jnp.float32),
            pltpu.VMEM((_HALF, 1, _BPW), jnp.float32),
            pltpu.SemaphoreType.DMA,
            pltpu.SemaphoreType.DMA,
        ],
    )
    out = kern(xp, tab)
    out = out.reshape(NUM_TABLES, VDJ_DIM, BATCH)
    return tuple(out[t].T for t in range(NUM_TABLES))


def kernel(x, W_v_alpha, W_j_alpha, W_v_beta, W_d_beta, W_j_beta):
    return _vdj_gather(x, W_v_alpha, W_j_alpha, W_v_beta, W_d_beta, W_j_beta)


# speedup vs baseline: 1.5398x; 1.5100x over previous
import jax
import jax.numpy as jnp
from jax import lax
from jax.experimental import pallas as pl
from jax.experimental.pallas import tpu as pltpu
from jax.experimental.pallas import tpu_sc as plsc

VDJ_DIM = 64
VOCAB = 1000
VOCAB_PAD = 1024
BATCH = 16384
NUM_TABLES = 5
TAB_WORDS = VDJ_DIM * VOCAB_PAD
HALF_D = 32
HALF_WORDS = HALF_D * VOCAB_PAD
N_HALF = NUM_TABLES * 2

_NC = 2
_NS = 16
_NW = _NC * _NS
_BPW = BATCH // _NW
_BC = _BPW // 128
_SLAB = _BC * 8 * 128
_L = 16


def _gather_body(xp_hbm, tab_hbm, o0, o1, o2, o3, o4,
                 idx_v, th0, th1, sl0, sl1, st0, st1, sw0, sw1):
    outs = (o0, o1, o2, o3, o4)
    ths = (th0, th1)
    sts = (st0, st1)
    slabs = (sl0, sl1)
    sws = (sw0, sw1)
    cid = lax.axis_index("c")
    sid = lax.axis_index("s")
    wid = sid * _NC + cid

    pltpu.sync_copy(xp_hbm.at[:, pl.ds(wid, 1)], idx_v)

    def load_half(h):
        return pltpu.async_copy(
            tab_hbm.at[pl.ds(h * HALF_WORDS, HALF_WORDS)],
            ths[h % 2], sts[h % 2])

    loads = [None] * N_HALF
    writes = [None, None]
    loads[0] = load_half(0)
    for h in range(N_HALF):
        t = h // 2
        if h + 1 < N_HALF:
            loads[h + 1] = load_half(h + 1)
        loads[h].wait()
        slab = slabs[h % 2]
        if writes[h % 2] is not None:
            for w in writes[h % 2]:
                w.wait()
        th = ths[h % 2]

        def chunk_body(j, _):
            xv = idx_v[t, 0, 0, pl.ds(j * _L, _L)]
            bc = j // 8
            k = j % 8
            dyn = bc * 1024 + k * _L
            for dl in range(HALF_D):
                v = plsc.load_gather(th, [xv + dl * VOCAB_PAD])
                off = (dl // 8) * _SLAB + (dl % 8) * 128
                slab[pl.ds(dyn + off, _L)] = v
            return 0

        lax.fori_loop(0, _BPW // _L, chunk_body, 0)

        half_d0 = (h % 2) * HALF_D
        ws = []
        for g in range(HALF_D // 8):
            dhi = half_d0 // 8 + g
            off = (dhi * 128 + wid * _BC) * 1024
            ws.append(pltpu.async_copy(
                slab.at[pl.ds(g * _SLAB, _SLAB)],
                outs[t].at[pl.ds(off, _SLAB)], sws[h % 2]))
        writes[h % 2] = ws
    for w in writes[0]:
        w.wait()
    for w in writes[1]:
        w.wait()


@jax.jit
def _vdj_gather(x, w0, w1, w2, w3, w4):
    xp = x.astype(jnp.int32).T.reshape(NUM_TABLES, _NW, 1, _BPW)
    pad = lambda w: jnp.pad(w.T, ((0, 0), (0, VOCAB_PAD - VOCAB))).reshape(-1)
    tab = jnp.concatenate([pad(w) for w in (w0, w1, w2, w3, w4)])

    kern = pl.kernel(
        _gather_body,
        out_type=tuple(
            jax.ShapeDtypeStruct((BATCH * VDJ_DIM,), jnp.float32)
            for _ in range(NUM_TABLES)
        ),
        mesh=plsc.VectorSubcoreMesh(core_axis_name="c", subcore_axis_name="s"),
        scratch_types=[
            pltpu.VMEM((NUM_TABLES, 1, 1, _BPW), jnp.int32),
            pltpu.VMEM((HALF_WORDS,), jnp.float32),
            pltpu.VMEM((HALF_WORDS,), jnp.float32),
            pltpu.VMEM((HALF_D // 8 * _SLAB,), jnp.float32),
            pltpu.VMEM((HALF_D // 8 * _SLAB,), jnp.float32),
            pltpu.SemaphoreType.DMA,
            pltpu.SemaphoreType.DMA,
            pltpu.SemaphoreType.DMA,
            pltpu.SemaphoreType.DMA,
        ],
        compiler_params=pltpu.CompilerParams(needs_layout_passes=False),
    )
    outs = kern(xp, tab)
    return tuple(
        o.reshape(8, 128, 8, 128).transpose(1, 3, 0, 2).reshape(BATCH, VDJ_DIM)
        for o in outs
    )


def kernel(x, W_v_alpha, W_j_alpha, W_v_beta, W_d_beta, W_j_beta):
    return _vdj_gather(x, W_v_alpha, W_j_alpha, W_v_beta, W_d_beta, W_j_beta)


# speedup vs baseline: 1.7816x; 1.1571x over previous
import jax
import jax.numpy as jnp
from jax import lax
from jax.experimental import pallas as pl
from jax.experimental.pallas import tpu as pltpu
from jax.experimental.pallas import tpu_sc as plsc

VDJ_DIM = 64
VOCAB = 1000
VOCAB_PAD = 1024
BATCH = 16384
NUM_TABLES = 5
TAB_WORDS = VDJ_DIM * VOCAB_PAD
HALF_D = 32
HALF_WORDS = HALF_D * VOCAB_PAD
N_HALF = NUM_TABLES * 2

_NC = 2
_NS = 16
_NW = _NC * _NS
_BPW = BATCH // _NW
_BC = _BPW // 128
_SLAB = _BC * 8 * 128
_L = 16


def _gather_body(xp_hbm, tab_hbm, o0, o1, o2, o3, o4,
                 idx_v, th0, th1, sl0, sl1, st0, st1, sw0, sw1):
    outs = (o0, o1, o2, o3, o4)
    ths = (th0, th1)
    sts = (st0, st1)
    slabs = (sl0, sl1)
    sws = (sw0, sw1)
    cid = lax.axis_index("c")
    sid = lax.axis_index("s")
    wid = sid * _NC + cid

    pltpu.sync_copy(xp_hbm.at[:, pl.ds(wid, 1)], idx_v)

    def load_half(h):
        return pltpu.async_copy(
            tab_hbm.at[pl.ds(h * HALF_WORDS, HALF_WORDS)],
            ths[h % 2], sts[h % 2])

    loads = [None] * N_HALF
    writes = [None, None]
    loads[0] = load_half(0)
    for h in range(N_HALF):
        t = h // 2
        if h + 1 < N_HALF:
            loads[h + 1] = load_half(h + 1)
        loads[h].wait()
        slab = slabs[h % 2]
        if writes[h % 2] is not None:
            for w in writes[h % 2]:
                w.wait()
        th = ths[h % 2]

        @plsc.parallel_loop(0, (_BPW // _L) * HALF_D, 1, unroll=8)
        def _gather_loop(i):
            j = i >> 5
            dl = i & (HALF_D - 1)
            xv = idx_v[t, 0, 0, pl.ds(j * _L, _L)]
            v = plsc.load_gather(th, [xv + dl * VOCAB_PAD])
            dyn = (j >> 3) * 1024 + (j & 7) * _L
            off = (dl >> 3) * _SLAB + (dl & 7) * 128
            slab[pl.ds(dyn + off, _L)] = v

        half_d0 = (h % 2) * HALF_D
        ws = []
        for g in range(HALF_D // 8):
            dhi = half_d0 // 8 + g
            off = (dhi * 128 + wid * _BC) * 1024
            ws.append(pltpu.async_copy(
                slab.at[pl.ds(g * _SLAB, _SLAB)],
                outs[t].at[pl.ds(off, _SLAB)], sws[h % 2]))
        writes[h % 2] = ws
    for w in writes[0]:
        w.wait()
    for w in writes[1]:
        w.wait()


@jax.jit
def _vdj_gather(x, w0, w1, w2, w3, w4):
    xp = x.astype(jnp.int32).T.reshape(NUM_TABLES, _NW, 1, _BPW)
    pad = lambda w: jnp.pad(w.T, ((0, 0), (0, VOCAB_PAD - VOCAB))).reshape(-1)
    tab = jnp.concatenate([pad(w) for w in (w0, w1, w2, w3, w4)])

    kern = pl.kernel(
        _gather_body,
        out_type=tuple(
            jax.ShapeDtypeStruct((BATCH * VDJ_DIM,), jnp.float32)
            for _ in range(NUM_TABLES)
        ),
        mesh=plsc.VectorSubcoreMesh(core_axis_name="c", subcore_axis_name="s"),
        scratch_types=[
            pltpu.VMEM((NUM_TABLES, 1, 1, _BPW), jnp.int32),
            pltpu.VMEM((HALF_WORDS,), jnp.float32),
            pltpu.VMEM((HALF_WORDS,), jnp.float32),
            pltpu.VMEM((HALF_D // 8 * _SLAB,), jnp.float32),
            pltpu.VMEM((HALF_D // 8 * _SLAB,), jnp.float32),
            pltpu.SemaphoreType.DMA,
            pltpu.SemaphoreType.DMA,
            pltpu.SemaphoreType.DMA,
            pltpu.SemaphoreType.DMA,
        ],
        compiler_params=pltpu.CompilerParams(needs_layout_passes=False),
    )
    outs = kern(xp, tab)
    return tuple(
        o.reshape(8, 128, 8, 128).transpose(1, 3, 0, 2).reshape(BATCH, VDJ_DIM)
        for o in outs
    )


def kernel(x, W_v_alpha, W_j_alpha, W_v_beta, W_d_beta, W_j_beta):
    return _vdj_gather(x, W_v_alpha, W_j_alpha, W_v_beta, W_d_beta, W_j_beta)
